# Initial kernel scaffold; baseline (speedup 1.0000x reference)
import jax
import jax.numpy as jnp
from jax.experimental import pallas as pl


def kernel(x, pool_idx):
    raise NotImplementedError("write your pallas kernel here")


# speedup vs baseline: 1.0365x; 1.0365x over previous
import functools

import jax
import jax.numpy as jnp
from jax import lax
from jax.experimental import pallas as pl
from jax.experimental.pallas import tpu as pltpu
from jax.experimental.pallas import tpu_sc as plsc

B, N, C, M = 8, 50000, 128, 25000
NC, NS = 2, 16
W = NC * NS
WPB = W // B
K = 128
STRIDE = 6256
NCHUNK = 49
TAIL_A = STRIDE - 48 * K
TAIL_B = M - 3 * STRIDE - 48 * K


def _gather_body(x_hbm, idx_hbm, out_hbm, idx_v, buf, sem):
    wid = lax.axis_index("s") * NC + lax.axis_index("c")
    b = wid // WPB
    q = wid % WPB
    mbase = q * STRIDE

    pltpu.sync_copy(idx_hbm.at[wid], idx_v)

    def chunk(ci, rows):
        pltpu.async_copy(x_hbm.at[b].at[idx_v.at[ci]], buf, sem).wait()
        pltpu.sync_copy(
            buf.at[pl.ds(0, rows)],
            out_hbm.at[b].at[pl.ds(mbase + ci * K, rows)],
        )

    def body(ci, carry):
        chunk(ci, K)
        return carry

    lax.fori_loop(0, NCHUNK - 1, body, 0)

    @pl.when(q < WPB - 1)
    def _():
        chunk(NCHUNK - 1, TAIL_A)

    @pl.when(q == WPB - 1)
    def _():
        chunk(NCHUNK - 1, TAIL_B)


@functools.partial(jax.jit, static_argnames=("interpret",))
def kernel(x, pool_idx, interpret=False):
    pos = jnp.arange(WPB)[:, None] * STRIDE + jnp.arange(NCHUNK * K)[None, :]
    pos = jnp.minimum(pos, M - 1)
    idx = pool_idx[:, pos].reshape(W, NCHUNK, K)

    mesh = plsc.VectorSubcoreMesh(
        core_axis_name="c", subcore_axis_name="s", num_cores=NC, num_subcores=NS
    )
    run = pl.kernel(
        _gather_body,
        out_type=jax.ShapeDtypeStruct((B, M, C), jnp.float32),
        mesh=mesh,
        scratch_types=[
            pltpu.VMEM((NCHUNK, K), jnp.int32),
            pltpu.VMEM((K, C), jnp.float32),
            pltpu.SemaphoreType.DMA,
        ],
        interpret=interpret,
    )
    return run(x, idx)


# speedup vs baseline: 1.2932x; 1.2476x over previous
import functools

import jax
import jax.numpy as jnp
from jax import lax
from jax.experimental import pallas as pl
from jax.experimental.pallas import tpu as pltpu
from jax.experimental.pallas import tpu_sc as plsc

B, N, C, M = 8, 50000, 128, 25000
NC, NS = 2, 16
W = NC * NS
WPB = W // B
K = 128
STRIDE = 6256
NCHUNK = 49
TAIL_A = STRIDE - 48 * K
TAIL_B = M - 3 * STRIDE - 48 * K


NBUF = 4
NFULL = NCHUNK - 1


def _gather_body(x_hbm, idx_hbm, out_hbm, idx_v, bufs, gsems, wsems):
    wid = lax.axis_index("s") * NC + lax.axis_index("c")
    b = wid // WPB
    q = wid % WPB
    mbase = q * STRIDE

    pltpu.sync_copy(idx_hbm.at[wid], idx_v)

    def start_gather(ci, j):
        pltpu.async_copy(x_hbm.at[b].at[idx_v.at[ci]], bufs.at[j], gsems.at[j])

    def start_write(ci, j):
        pltpu.async_copy(
            bufs.at[j], out_hbm.at[b].at[pl.ds(mbase + ci * K, K)], wsems.at[j]
        )

    def drain(sems, j, rows=K):
        pltpu.make_async_copy(
            x_hbm.at[b].at[pl.ds(0, rows)], bufs.at[j, pl.ds(0, rows)], sems.at[j]
        ).wait()

    for j in range(NBUF):
        start_gather(j, j)

    def body(i, carry):
        for j in range(NBUF):
            drain(gsems, j)
            start_write(NBUF * i + j, j)
        for j in range(NBUF):
            nxt = NBUF * i + j + NBUF

            @pl.when(nxt <= NFULL)
            def _():
                drain(wsems, j)
                start_gather(nxt, j)

        return carry

    lax.fori_loop(0, NFULL // NBUF, body, 0)

    drain(gsems, 0)

    @pl.when(q < WPB - 1)
    def _():
        pltpu.sync_copy(
            bufs.at[0, pl.ds(0, TAIL_A)],
            out_hbm.at[b].at[pl.ds(mbase + NFULL * K, TAIL_A)],
        )

    @pl.when(q == WPB - 1)
    def _():
        pltpu.sync_copy(
            bufs.at[0, pl.ds(0, TAIL_B)],
            out_hbm.at[b].at[pl.ds(mbase + NFULL * K, TAIL_B)],
        )

    for j in range(1, NBUF):
        drain(wsems, j)


@functools.partial(jax.jit, static_argnames=("interpret",))
def kernel(x, pool_idx, interpret=False):
    pos = jnp.arange(WPB)[:, None] * STRIDE + jnp.arange(NCHUNK * K)[None, :]
    pos = jnp.minimum(pos, M - 1)
    idx = pool_idx[:, pos].reshape(W, NCHUNK, K)

    mesh = plsc.VectorSubcoreMesh(
        core_axis_name="c", subcore_axis_name="s", num_cores=NC, num_subcores=NS
    )
    run = pl.kernel(
        _gather_body,
        out_type=jax.ShapeDtypeStruct((B, M, C), jnp.float32),
        mesh=mesh,
        scratch_types=[
            pltpu.VMEM((NCHUNK, K), jnp.int32),
            pltpu.VMEM((NBUF, K, C), jnp.float32),
            pltpu.SemaphoreType.DMA((NBUF,)),
            pltpu.SemaphoreType.DMA((NBUF,)),
        ],
        interpret=interpret,
    )
    return run(x, idx)


# speedup vs baseline: 1.7321x; 1.3394x over previous
import functools

import jax
import jax.numpy as jnp
from jax import lax
from jax.experimental import pallas as pl
from jax.experimental.pallas import tpu as pltpu
from jax.experimental.pallas import tpu_sc as plsc

B, N, C, M = 8, 50000, 128, 25000
NC, NS = 2, 16
W = NC * NS
WPB = W // B
K = 128
STRIDE = 6256
NCHUNK = 49
TAIL_A = STRIDE - 48 * K
TAIL_B = M - 3 * STRIDE - 48 * K


NBUF = 4
NFULL = NCHUNK - 1


def _gather_body(x_hbm, idx_hbm, out_hbm, idx_v, bufs, gsems, wsems):
    wid = lax.axis_index("s") * NC + lax.axis_index("c")
    b = wid // WPB
    q = wid % WPB
    mbase = q * STRIDE

    pltpu.sync_copy(idx_hbm.at[pl.ds(b * M + q * STRIDE, NCHUNK * K)], idx_v)

    def start_gather(ci, j):
        pltpu.async_copy(
            x_hbm.at[b].at[idx_v.at[pl.ds(ci * K, K)]], bufs.at[j], gsems.at[j]
        )

    def start_write(ci, j):
        pltpu.async_copy(
            bufs.at[j], out_hbm.at[b].at[pl.ds(mbase + ci * K, K)], wsems.at[j]
        )

    def drain(sems, j, rows=K):
        pltpu.make_async_copy(
            x_hbm.at[b].at[pl.ds(0, rows)], bufs.at[j, pl.ds(0, rows)], sems.at[j]
        ).wait()

    for j in range(NBUF):
        start_gather(j, j)

    def body(i, carry):
        for j in range(NBUF):
            drain(gsems, j)
            start_write(NBUF * i + j, j)
        for j in range(NBUF):
            nxt = NBUF * i + j + NBUF

            @pl.when(nxt <= NFULL)
            def _():
                drain(wsems, j)
                start_gather(nxt, j)

        return carry

    lax.fori_loop(0, NFULL // NBUF, body, 0)

    drain(gsems, 0)

    @pl.when(q < WPB - 1)
    def _():
        pltpu.sync_copy(
            bufs.at[0, pl.ds(0, TAIL_A)],
            out_hbm.at[b].at[pl.ds(mbase + NFULL * K, TAIL_A)],
        )

    @pl.when(q == WPB - 1)
    def _():
        pltpu.sync_copy(
            bufs.at[0, pl.ds(0, TAIL_B)],
            out_hbm.at[b].at[pl.ds(mbase + NFULL * K, TAIL_B)],
        )

    for j in range(1, NBUF):
        drain(wsems, j)


@functools.partial(jax.jit, static_argnames=("interpret",))
def kernel(x, pool_idx, interpret=False):
    idx = jnp.pad(pool_idx.reshape(B * M), (0, 64))

    mesh = plsc.VectorSubcoreMesh(
        core_axis_name="c", subcore_axis_name="s", num_cores=NC, num_subcores=NS
    )
    run = pl.kernel(
        _gather_body,
        out_type=jax.ShapeDtypeStruct((B, M, C), jnp.float32),
        mesh=mesh,
        scratch_types=[
            pltpu.VMEM((NCHUNK * K,), jnp.int32),
            pltpu.VMEM((NBUF, K, C), jnp.float32),
            pltpu.SemaphoreType.DMA((NBUF,)),
            pltpu.SemaphoreType.DMA((NBUF,)),
        ],
        interpret=interpret,
    )
    return run(x, idx)


# speedup vs baseline: 1.7816x; 1.0286x over previous
import functools

import jax
import jax.numpy as jnp
from jax import lax
from jax.experimental import pallas as pl
from jax.experimental.pallas import tpu as pltpu
from jax.experimental.pallas import tpu_sc as plsc

B, N, C, M = 8, 50000, 128, 25000
NC, NS = 2, 16
W = NC * NS
WPB = W // B
K = 128
STRIDE = 6256
NCHUNK = 49
TAIL_A = STRIDE - 48 * K
TAIL_B = M - 3 * STRIDE - 48 * K


NBUF = 6
NFULL = NCHUNK - 1


def _gather_body(x_hbm, idx_hbm, out_hbm, idx_v, bufs, gsems, wsems):
    wid = lax.axis_index("s") * NC + lax.axis_index("c")
    b = wid // WPB
    q = wid % WPB
    mbase = q * STRIDE

    pltpu.sync_copy(idx_hbm.at[pl.ds(b * M + q * STRIDE, NCHUNK * K)], idx_v)

    def start_gather(ci, j):
        pltpu.async_copy(
            x_hbm.at[b].at[idx_v.at[pl.ds(ci * K, K)]], bufs.at[j], gsems.at[j]
        )

    def start_write(ci, j):
        pltpu.async_copy(
            bufs.at[j], out_hbm.at[b].at[pl.ds(mbase + ci * K, K)], wsems.at[j]
        )

    def drain(sems, j, rows=K):
        pltpu.make_async_copy(
            x_hbm.at[b].at[pl.ds(0, rows)], bufs.at[j, pl.ds(0, rows)], sems.at[j]
        ).wait()

    for j in range(NBUF):
        start_gather(j, j)

    def body(i, carry):
        for j in range(NBUF):
            drain(gsems, j)
            start_write(NBUF * i + j, j)
        for j in range(NBUF):
            nxt = NBUF * i + j + NBUF

            @pl.when(nxt <= NFULL)
            def _():
                drain(wsems, j)
                start_gather(nxt, j)

        return carry

    lax.fori_loop(0, NFULL // NBUF, body, 0)

    drain(gsems, 0)

    @pl.when(q < WPB - 1)
    def _():
        pltpu.sync_copy(
            bufs.at[0, pl.ds(0, TAIL_A)],
            out_hbm.at[b].at[pl.ds(mbase + NFULL * K, TAIL_A)],
        )

    @pl.when(q == WPB - 1)
    def _():
        pltpu.sync_copy(
            bufs.at[0, pl.ds(0, TAIL_B)],
            out_hbm.at[b].at[pl.ds(mbase + NFULL * K, TAIL_B)],
        )

    for j in range(1, NBUF):
        drain(wsems, j)


@functools.partial(jax.jit, static_argnames=("interpret",))
def kernel(x, pool_idx, interpret=False):
    idx = jnp.pad(pool_idx.reshape(B * M), (0, 64))

    mesh = plsc.VectorSubcoreMesh(
        core_axis_name="c", subcore_axis_name="s", num_cores=NC, num_subcores=NS
    )
    run = pl.kernel(
        _gather_body,
        out_type=jax.ShapeDtypeStruct((B, M, C), jnp.float32),
        mesh=mesh,
        scratch_types=[
            pltpu.VMEM((NCHUNK * K,), jnp.int32),
            pltpu.VMEM((NBUF, K, C), jnp.float32),
            pltpu.SemaphoreType.DMA((NBUF,)),
            pltpu.SemaphoreType.DMA((NBUF,)),
        ],
        interpret=interpret,
    )
    return run(x, idx)
